# baseline (speedup 1.0000x reference)
import functools

import jax
import jax.numpy as jnp
from jax import lax
from jax.experimental import pallas as pl
from jax.experimental.pallas import tpu as pltpu
from jax.experimental.pallas import tpu_sc as plsc

MAX_LEN = 8192
D_MODEL = 768
BATCH = 4
SEQ = 8192
B_TOT = BATCH * SEQ
NW = 32
NS = 16
B_PER_W = B_TOT // NW
CHUNK = 16
NCHUNK = B_PER_W // CHUNK
NBUF = 4
SBUF = 2

_mesh = plsc.VectorSubcoreMesh(core_axis_name="c", subcore_axis_name="s")


@functools.partial(
    pl.kernel,
    mesh=_mesh,
    out_type=jax.ShapeDtypeStruct((BATCH, SEQ, 1, D_MODEL), jnp.float32),
    scratch_types=[
        pltpu.VMEM((NCHUNK, CHUNK), jnp.int32),
        pltpu.VMEM((NBUF, CHUNK, 1, D_MODEL), jnp.float32),
        pltpu.VMEM_SHARED((NS, SBUF, CHUNK, 1, D_MODEL), jnp.float32),
        [pltpu.SemaphoreType.DMA] * NBUF,
        [pltpu.SemaphoreType.DMA] * NBUF,
        [pltpu.SemaphoreType.DMA] * SBUF,
    ],
)
def _gather_rows(idx_hbm, table_hbm, out_hbm, idx_v, bufs, spb, gsems, tsems, hsems):
    sid = lax.axis_index("s")
    wid = sid * 2 + lax.axis_index("c")
    batch = wid // (NW // BATCH)
    seq_base = (wid % (NW // BATCH)) * B_PER_W
    pltpu.sync_copy(idx_hbm.at[wid], idx_v)

    gcop = [None] * NBUF
    tcop = [None] * NBUF
    hcop = [None] * SBUF
    for b in range(NBUF - 1):
        gcop[b] = pltpu.async_copy(table_hbm.at[idx_v.at[b]], bufs.at[b], gsems[b])
    for c in range(NCHUNK):
        b = c % NBUF
        if c > 0:
            pb = (c - 1) % NBUF
            tcop[pb].wait()
            hcop[(c - 1) % SBUF] = pltpu.async_copy(
                spb.at[sid, (c - 1) % SBUF],
                out_hbm.at[batch, pl.ds(seq_base + (c - 1) * CHUNK, CHUNK)],
                hsems[(c - 1) % SBUF],
            )
        nc = c + NBUF - 1
        if nc < NCHUNK:
            fb = nc % NBUF
            gcop[fb] = pltpu.async_copy(
                table_hbm.at[idx_v.at[nc]], bufs.at[fb], gsems[fb]
            )
        gcop[b].wait()
        sb = c % SBUF
        if c >= SBUF:
            hcop[sb].wait()
        tcop[b] = pltpu.async_copy(bufs.at[b], spb.at[sid, sb], tsems[b])
    last = NCHUNK - 1
    tcop[last % NBUF].wait()
    hcop[last % SBUF] = pltpu.async_copy(
        spb.at[sid, last % SBUF],
        out_hbm.at[batch, pl.ds(seq_base + last * CHUNK, CHUNK)],
        hsems[last % SBUF],
    )
    for c in range(NCHUNK - SBUF, NCHUNK):
        hcop[c % SBUF].wait()


def kernel(position_ids, pe):
    idx = position_ids.reshape(NW, NCHUNK, CHUNK).astype(jnp.int32)
    return _gather_rows(idx, pe)


# speedup vs baseline: 1.0464x; 1.0464x over previous
import functools

import jax
import jax.numpy as jnp
from jax import lax
from jax.experimental import pallas as pl
from jax.experimental.pallas import tpu as pltpu
from jax.experimental.pallas import tpu_sc as plsc

MAX_LEN = 8192
D_MODEL = 768
BATCH = 4
SEQ = 8192
B_TOT = BATCH * SEQ
NW = 32
B_PER_W = B_TOT // NW
CHUNK = 32
NCHUNK = B_PER_W // CHUNK
NBUF = 5

_mesh = plsc.VectorSubcoreMesh(core_axis_name="c", subcore_axis_name="s")


@functools.partial(
    pl.kernel,
    mesh=_mesh,
    out_type=jax.ShapeDtypeStruct((BATCH, SEQ, 1, D_MODEL), jnp.float32),
    scratch_types=[
        pltpu.VMEM((NCHUNK, CHUNK), jnp.int32),
        pltpu.VMEM((NBUF, CHUNK, 1, D_MODEL), jnp.float32),
        [pltpu.SemaphoreType.DMA] * NBUF,
        [pltpu.SemaphoreType.DMA] * NBUF,
    ],
)
def _gather_rows(idx_hbm, table_hbm, out_hbm, idx_v, bufs, gsems, ssems):
    wid = lax.axis_index("s") * 2 + lax.axis_index("c")
    batch = wid // (NW // BATCH)
    seq_base = (wid % (NW // BATCH)) * B_PER_W
    pltpu.sync_copy(idx_hbm.at[wid], idx_v)

    gcop = [None] * NBUF
    scop = [None] * NBUF
    for b in range(NBUF - 1):
        gcop[b] = pltpu.async_copy(table_hbm.at[idx_v.at[b]], bufs.at[b], gsems[b])
    for c in range(NCHUNK):
        b = c % NBUF
        nc = c + NBUF - 1
        if nc < NCHUNK:
            fb = nc % NBUF
            if c > 0:
                scop[fb].wait()
            gcop[fb] = pltpu.async_copy(
                table_hbm.at[idx_v.at[nc]], bufs.at[fb], gsems[fb]
            )
        gcop[b].wait()
        scop[b] = pltpu.async_copy(
            bufs.at[b],
            out_hbm.at[batch, pl.ds(seq_base + c * CHUNK, CHUNK)],
            ssems[b],
        )
    for c in range(max(0, NCHUNK - NBUF + 1), NCHUNK):
        scop[c % NBUF].wait()


def kernel(position_ids, pe):
    idx = position_ids.reshape(NW, NCHUNK, CHUNK).astype(jnp.int32)
    return _gather_rows(idx, pe)
